# Initial kernel scaffold; baseline (speedup 1.0000x reference)
import jax
import jax.numpy as jnp
from jax.experimental import pallas as pl


def kernel(x, edge_index, edge_attr, batch, fc1_w, fc1_b, fc2_w, fc2_b, root, conv_b, pre_w, pre_b, out_w, out_b):
    raise NotImplementedError("write your pallas kernel here")


# speedup vs baseline: 2.5790x; 2.5790x over previous
import functools

import jax
import jax.numpy as jnp
from jax import lax
from jax.experimental import pallas as pl
from jax.experimental.pallas import tpu as pltpu
from jax.experimental.pallas import tpu_sc as plsc

N = 10000
E = 160000
D_IN = 128
D_EDGE = 16
HID = 16
OUT = 32
G = 64

NPAD = 10240
ROWS_PER_TILE = NPAD // 16
CHUNK = 128
NCHUNKS = E // CHUNK
NC = 2
NS = 16
NW = NC * NS
ITERS = (NCHUNKS + NW - 1) // NW

BE = 640
BN = 1000


def _sc_mesh():
    return plsc.VectorSubcoreMesh(
        core_axis_name="c", subcore_axis_name="s", num_cores=NC, num_subcores=NS
    )


def _gather_body(x_hbm, src_hbm, out_hbm, idx_v, rows_v, sem):
    c = lax.axis_index("c")
    s = lax.axis_index("s")
    wid = s * NC + c

    def body(i, carry):
        chunk = jnp.minimum(wid + i * NW, NCHUNKS - 1)
        base = chunk * CHUNK
        pltpu.sync_copy(src_hbm.at[pl.ds(base, CHUNK)], idx_v)
        pltpu.async_copy(x_hbm.at[idx_v], rows_v, sem).wait()
        pltpu.sync_copy(rows_v, out_hbm.at[pl.ds(base, CHUNK)])
        return carry

    lax.fori_loop(0, ITERS, body, 0)


def _gather(x, src):
    kfn = functools.partial(
        pl.kernel,
        out_type=jax.ShapeDtypeStruct((E, D_IN), jnp.float32),
        mesh=_sc_mesh(),
        scratch_types=[
            pltpu.VMEM((CHUNK,), jnp.int32),
            pltpu.VMEM((CHUNK, D_IN), jnp.float32),
            pltpu.SemaphoreType.DMA,
        ],
    )
    return kfn(_gather_body)(x, src)


def _scatter_body(dst_hbm, msg_hbm, zeros_hbm, out_hbm, idx_v, msg_v, acc, sem):
    c = lax.axis_index("c")
    s = lax.axis_index("s")
    wid = s * NC + c

    pltpu.sync_copy(zeros_hbm, acc.at[pl.ds(s * ROWS_PER_TILE, ROWS_PER_TILE)])
    plsc.subcore_barrier()

    def body(i, carry):
        chunk = wid + i * NW

        @pl.when(chunk < NCHUNKS)
        def _():
            base = chunk * CHUNK
            pltpu.sync_copy(dst_hbm.at[pl.ds(base, CHUNK)], idx_v.at[0])
            pltpu.sync_copy(msg_hbm.at[pl.ds(base, CHUNK)], msg_v)
            pltpu.sync_copy(msg_v, acc.at[idx_v.at[0]], add=True)

        return carry

    lax.fori_loop(0, ITERS, body, 0)
    plsc.subcore_barrier()
    pltpu.sync_copy(
        acc.at[pl.ds(s * ROWS_PER_TILE, ROWS_PER_TILE)],
        out_hbm.at[c, pl.ds(s * ROWS_PER_TILE, ROWS_PER_TILE)],
    )


def _scatter(dst, msg2):
    zeros = jnp.zeros((ROWS_PER_TILE, 2 * HID), jnp.float32)
    kfn = functools.partial(
        pl.kernel,
        out_type=jax.ShapeDtypeStruct((NC, NPAD, 2 * HID), jnp.float32),
        mesh=_sc_mesh(),
        scratch_types=[
            pltpu.VMEM((1, CHUNK), jnp.int32),
            pltpu.VMEM((CHUNK, 2 * HID), jnp.float32),
            pltpu.VMEM_SHARED((NPAD, 2 * HID), jnp.float32),
            pltpu.SemaphoreType.DMA,
        ],
        compiler_params=pltpu.CompilerParams(use_tc_tiling_on_sc=False),
    )
    return kfn(_scatter_body)(dst, msg2, zeros)


def _edge_kernel(ea_ref, xs_ref, fc1t_ref, fc1b_ref, fc2p_ref, bmat_ref,
                 sel_ref, out_ref):
    ea = ea_ref[...]
    h = jnp.maximum(
        jnp.dot(ea, fc1t_ref[...], preferred_element_type=jnp.float32)
        + fc1b_ref[...],
        0.0,
    )
    w2 = jnp.dot(
        h.astype(jnp.bfloat16), fc2p_ref[...], preferred_element_type=jnp.float32
    ).astype(jnp.bfloat16)
    xs = xs_ref[...]
    xst = jnp.concatenate([xs.astype(jnp.bfloat16)] * HID, axis=1)
    msg = jnp.dot(w2 * xst, sel_ref[...], preferred_element_type=jnp.float32)
    msg = msg + jnp.dot(xs, bmat_ref[...], preferred_element_type=jnp.float32)
    ones = jnp.ones((BE, 1), jnp.float32)
    pad = jnp.zeros((BE, HID - 1), jnp.float32)
    out_ref[...] = jnp.concatenate([msg, ones, pad], axis=1)


def _edge_stage(ea, x_src, fc1t, fc1b, fc2p, bmat, sel):
    grid = (E // BE,)
    return pl.pallas_call(
        _edge_kernel,
        grid=grid,
        in_specs=[
            pl.BlockSpec((BE, D_EDGE), lambda i: (i, 0)),
            pl.BlockSpec((BE, D_IN), lambda i: (i, 0)),
            pl.BlockSpec((D_EDGE, 128), lambda i: (0, 0)),
            pl.BlockSpec((1, 128), lambda i: (0, 0)),
            pl.BlockSpec((128, HID * D_IN), lambda i: (0, 0)),
            pl.BlockSpec((D_IN, HID), lambda i: (0, 0)),
            pl.BlockSpec((HID * D_IN, HID), lambda i: (0, 0)),
        ],
        out_specs=pl.BlockSpec((BE, 2 * HID), lambda i: (i, 0)),
        out_shape=jax.ShapeDtypeStruct((E, 2 * HID), jnp.float32),
    )(ea, x_src, fc1t, fc1b, fc2p, bmat, sel)


def _final_kernel(
    p_ref, x_ref, batch_ref, root_ref, convb_ref, prew_ref, preb_ref,
    outw_ref, outb_ref, out_ref, gs_acc, gc_acc,
):
    i = pl.program_id(0)
    p = p_ref[...]
    srows = p[0] + p[1]
    summed = srows[:, :HID]
    cnt = srows[:, HID:HID + 1]
    aggr = summed / jnp.maximum(cnt, 1.0)
    xr = jnp.dot(x_ref[...], root_ref[...], preferred_element_type=jnp.float32)
    xh = jnp.maximum(aggr + xr + convb_ref[...], 0.0)
    xp = jnp.maximum(
        jnp.dot(xh, prew_ref[...], preferred_element_type=jnp.float32)
        + preb_ref[...],
        0.0,
    )
    b2 = batch_ref[...].reshape(1, BN)
    gid = lax.broadcasted_iota(jnp.int32, (G, 1), 0)
    mask = (b2 == gid).astype(jnp.float32)
    gs = jnp.dot(mask, xp, preferred_element_type=jnp.float32)
    gc = jnp.sum(mask, axis=1, keepdims=True)

    @pl.when(i == 0)
    def _():
        gs_acc[...] = jnp.zeros_like(gs_acc)
        gc_acc[...] = jnp.zeros_like(gc_acc)

    gs_acc[...] += gs
    gc_acc[...] += gc

    @pl.when(i == pl.num_programs(0) - 1)
    def _():
        pooled = gs_acc[...] / jnp.maximum(gc_acc[...], 1.0)
        out_ref[...] = jnp.maximum(
            jnp.dot(pooled, outw_ref[...], preferred_element_type=jnp.float32)
            + outb_ref[...],
            0.0,
        )


def _final_stage(p, x, batch3, root, convb, prew, preb, outw, outb):
    grid = (N // BN,)
    return pl.pallas_call(
        _final_kernel,
        grid=grid,
        in_specs=[
            pl.BlockSpec((NC, BN, 2 * HID), lambda i: (0, i, 0)),
            pl.BlockSpec((BN, D_IN), lambda i: (i, 0)),
            pl.BlockSpec((1, 1, BN), lambda i: (i, 0, 0)),
            pl.BlockSpec((D_IN, HID), lambda i: (0, 0)),
            pl.BlockSpec((1, HID), lambda i: (0, 0)),
            pl.BlockSpec((HID, HID), lambda i: (0, 0)),
            pl.BlockSpec((1, HID), lambda i: (0, 0)),
            pl.BlockSpec((HID, OUT), lambda i: (0, 0)),
            pl.BlockSpec((1, OUT), lambda i: (0, 0)),
        ],
        out_specs=pl.BlockSpec((G, OUT), lambda i: (0, 0)),
        out_shape=jax.ShapeDtypeStruct((G, OUT), jnp.float32),
        scratch_shapes=[
            pltpu.VMEM((G, HID), jnp.float32),
            pltpu.VMEM((G, 1), jnp.float32),
        ],
    )(p, x, batch3, root, convb, prew, preb, outw, outb)


def kernel(x, edge_index, edge_attr, batch, fc1_w, fc1_b, fc2_w, fc2_b,
           root, conv_b, pre_w, pre_b, out_w, out_b):
    src = edge_index[0].astype(jnp.int32)
    dst = edge_index[1].astype(jnp.int32)
    batch3 = batch.astype(jnp.int32).reshape(N // BN, 1, BN)

    fc1t = fc1_w.T
    fc1b = fc1_b.reshape(1, 128)
    fc2p = (
        fc2_w.T.reshape(128, D_IN, HID)
        .transpose(0, 2, 1)
        .reshape(128, HID * D_IN)
        .astype(jnp.bfloat16)
    )
    bmat = fc2_b.reshape(D_IN, HID)
    sel = jnp.repeat(jnp.eye(HID, dtype=jnp.bfloat16), D_IN, axis=0)

    x_src = _gather(x, src)
    msg2 = _edge_stage(edge_attr, x_src, fc1t, fc1b, fc2p, bmat, sel)
    partials = _scatter(dst, msg2)
    p = partials[:, :N, :]
    return _final_stage(
        p, x, batch3, root, conv_b.reshape(1, HID), pre_w.T,
        pre_b.reshape(1, HID), out_w.T, out_b.reshape(1, OUT),
    )
